# baseline (speedup 1.0000x reference)
import functools

import jax
import jax.numpy as jnp
from jax import lax
from jax.experimental import pallas as pl
from jax.experimental.pallas import tpu as pltpu
from jax.experimental.pallas import tpu_sc as plsc

VOCAB = 1000000
D = 64
W = 128
TBLK = 512
TGRID = 977
VROWS = TBLK * TGRID
SEQ = 200
PSEQ = 256
NFULL = SEQ // 16
TAIL = SEQ - 16 * NFULL
B = 4096
H = 100
C = 10

_info = plsc.get_sparse_core_info()
NC = _info.num_cores
NS = _info.num_subcores
NW = NC * NS
BPW = B // NW
NV = D // 16


def _pack_body(lo_ref, hi_ref, out_ref):
    out_ref[:, 0:D] = jnp.transpose(lo_ref[...])
    out_ref[:, D:W] = jnp.transpose(hi_ref[...])


def _pack_table(embT):
    return pl.pallas_call(
        _pack_body,
        grid=(TGRID,),
        in_specs=[
            pl.BlockSpec((D, TBLK), lambda j: (0, j)),
            pl.BlockSpec((D, TBLK), lambda j: (0, j + TGRID)),
        ],
        out_specs=pl.BlockSpec((TBLK, W), lambda j: (j, 0)),
        out_shape=jax.ShapeDtypeStruct((VROWS, W), jnp.float32),
    )(embT, embT)


def _bag_body(idx_hbm, par_hbm, table_hbm, out_hbm,
              idx_v, par_v, rows_v, out_v, sem0, sem1):
    wid = lax.axis_index("s") * NC + lax.axis_index("c")
    base = wid * BPW

    pltpu.sync_copy(idx_hbm.at[pl.ds(base * PSEQ, BPW * PSEQ)], idx_v)
    pltpu.sync_copy(par_hbm.at[pl.ds(base * PSEQ, BPW * PSEQ)], par_v)

    def issue(b, buf, sem):
        pltpu.async_copy(
            table_hbm.at[idx_v.at[pl.ds(b * PSEQ, 128)]],
            rows_v.at[buf, pl.ds(0, 128)], sem)
        pltpu.async_copy(
            table_hbm.at[idx_v.at[pl.ds(b * PSEQ + 128, 72)]],
            rows_v.at[buf, pl.ds(128, 72)], sem)

    def drain(buf, sem):
        pltpu.make_async_copy(
            table_hbm.at[pl.ds(0, 128)], rows_v.at[buf, pl.ds(0, 128)],
            sem).wait()
        pltpu.make_async_copy(
            table_hbm.at[pl.ds(0, 72)], rows_v.at[buf, pl.ds(128, 72)],
            sem).wait()

    def accumulate(b, buf):
        def rows16(g, accs, n=16):
            pv = par_v[pl.ds(b * PSEQ + g * 16, 16)]
            for j in range(n):
                r = g * 16 + j
                hi_half = pv[j] != 0
                new = []
                for c in range(NV):
                    lo = rows_v[buf, r, pl.ds(c * 16, 16)]
                    hi = rows_v[buf, r, pl.ds(D + c * 16, 16)]
                    new.append(accs[c] + jnp.where(hi_half, hi, lo))
                accs = tuple(new)
            return accs

        zero = jnp.zeros((16,), jnp.float32)
        accs = lax.fori_loop(0, NFULL, rows16, (zero,) * NV)
        accs = rows16(NFULL, accs, n=TAIL)
        inv = jnp.float32(1.0 / SEQ)
        for c in range(NV):
            out_v[pl.ds(b * D + c * 16, 16)] = accs[c] * inv

    issue(0, 0, sem0)

    def body(i, _):
        b0 = 2 * i
        b1 = 2 * i + 1
        issue(b1, 1, sem1)
        drain(0, sem0)
        accumulate(b0, 0)

        @pl.when(b1 + 1 < BPW)
        def _():
            issue(b1 + 1, 0, sem0)

        drain(1, sem1)
        accumulate(b1, 1)
        return 0

    lax.fori_loop(0, BPW // 2, body, 0)
    pltpu.sync_copy(out_v, out_hbm.at[pl.ds(base * D, BPW * D)])


def _bag_means(idxp, parp, table2):
    mesh = plsc.VectorSubcoreMesh(core_axis_name="c", subcore_axis_name="s")
    f = functools.partial(
        pl.kernel,
        mesh=mesh,
        out_type=jax.ShapeDtypeStruct((B * D,), jnp.float32),
        scratch_types=[
            pltpu.VMEM((BPW * PSEQ,), jnp.int32),
            pltpu.VMEM((BPW * PSEQ,), jnp.int32),
            pltpu.VMEM((2, SEQ, W), jnp.float32),
            pltpu.VMEM((BPW * D,), jnp.float32),
            pltpu.SemaphoreType.DMA,
            pltpu.SemaphoreType.DMA,
        ],
    )(_bag_body)
    return f(idxp, parp, table2)


def _mlp_body(bag_ref, w1_ref, b1_ref, w2_ref, b2_ref, out_ref):
    h = jnp.dot(bag_ref[...], w1_ref[...], preferred_element_type=jnp.float32)
    h = jnp.maximum(h + b1_ref[...], 0.0)
    out_ref[...] = (
        jnp.dot(h, w2_ref[...], preferred_element_type=jnp.float32)
        + b2_ref[...]
    )


def _mlp(bag, W1, b1, W2, b2):
    return pl.pallas_call(
        _mlp_body,
        out_shape=jax.ShapeDtypeStruct((B, C), jnp.float32),
    )(bag, W1, b1.reshape(1, H), W2, b2.reshape(1, C))


def kernel(inputX, emb, W1, b1, W2, b2):
    idx = inputX.astype(jnp.int32)
    idxp = jnp.pad(idx % VROWS, ((0, 0), (0, PSEQ - SEQ))).reshape(-1)
    parp = jnp.pad(
        (idx >= VROWS).astype(jnp.int32), ((0, 0), (0, PSEQ - SEQ))
    ).reshape(-1)
    table2 = _pack_table(emb.T)
    bag = _bag_means(idxp, parp, table2).reshape(B, D)
    return _mlp(bag, W1, b1, W2, b2)


# speedup vs baseline: 2.0225x; 2.0225x over previous
import functools

import jax
import jax.numpy as jnp
from jax import lax
from jax.experimental import pallas as pl
from jax.experimental.pallas import tpu as pltpu
from jax.experimental.pallas import tpu_sc as plsc

VOCAB = 1000000
D = 64
W = 128
TBLK = 8192
TGRID = 62
VROWS = TBLK * TGRID
SEQ = 200
PSEQ = 256
NFULL = SEQ // 16
TAIL = SEQ - 16 * NFULL
B = 4096
H = 100
C = 10

_info = plsc.get_sparse_core_info()
NC = _info.num_cores
NS = _info.num_subcores
NW = NC * NS
BPW = B // NW
NV = D // 16


def _pack_body(lo_ref, hi_ref, out_ref):
    out_ref[:, 0:D] = jnp.transpose(lo_ref[...])
    out_ref[:, D:W] = jnp.transpose(hi_ref[...])


def _pack_table(embT):
    return pl.pallas_call(
        _pack_body,
        grid=(TGRID,),
        in_specs=[
            pl.BlockSpec((D, TBLK), lambda j: (0, j)),
            pl.BlockSpec(
                (D, TBLK),
                lambda j: (0, jnp.minimum(j + TGRID, VOCAB // TBLK)),
            ),
        ],
        out_specs=pl.BlockSpec((TBLK, W), lambda j: (j, 0)),
        out_shape=jax.ShapeDtypeStruct((VROWS, W), jnp.float32),
    )(embT, embT)


def _bag_body(idx_hbm, par_hbm, table_hbm, out_hbm,
              idx_v, par_v, rows_v, out_v, sem0, sem1):
    wid = lax.axis_index("s") * NC + lax.axis_index("c")
    base = wid * BPW

    pltpu.sync_copy(idx_hbm.at[pl.ds(base * PSEQ, BPW * PSEQ)], idx_v)
    pltpu.sync_copy(par_hbm.at[pl.ds(base * PSEQ, BPW * PSEQ)], par_v)

    def issue(b, buf, sem):
        pltpu.async_copy(
            table_hbm.at[idx_v.at[pl.ds(b * PSEQ, 128)]],
            rows_v.at[buf, pl.ds(0, 128)], sem)
        pltpu.async_copy(
            table_hbm.at[idx_v.at[pl.ds(b * PSEQ + 128, 72)]],
            rows_v.at[buf, pl.ds(128, 72)], sem)

    def drain(buf, sem):
        pltpu.make_async_copy(
            table_hbm.at[pl.ds(0, 128)], rows_v.at[buf, pl.ds(0, 128)],
            sem).wait()
        pltpu.make_async_copy(
            table_hbm.at[pl.ds(0, 72)], rows_v.at[buf, pl.ds(128, 72)],
            sem).wait()

    def accumulate(b, buf):
        def rows16(g, accs, n=16):
            pv = par_v[pl.ds(b * PSEQ + g * 16, 16)]
            for j in range(n):
                r = g * 16 + j
                hi_half = pv[j] != 0
                new = []
                for c in range(NV):
                    lo = rows_v[buf, r, pl.ds(c * 16, 16)]
                    hi = rows_v[buf, r, pl.ds(D + c * 16, 16)]
                    new.append(accs[c] + jnp.where(hi_half, hi, lo))
                accs = tuple(new)
            return accs

        zero = jnp.zeros((16,), jnp.float32)
        accs = lax.fori_loop(0, NFULL, rows16, (zero,) * NV)
        accs = rows16(NFULL, accs, n=TAIL)
        inv = jnp.float32(1.0 / SEQ)
        for c in range(NV):
            out_v[pl.ds(b * D + c * 16, 16)] = accs[c] * inv

    issue(0, 0, sem0)

    def body(i, _):
        b0 = 2 * i
        b1 = 2 * i + 1
        issue(b1, 1, sem1)
        drain(0, sem0)
        accumulate(b0, 0)

        @pl.when(b1 + 1 < BPW)
        def _():
            issue(b1 + 1, 0, sem0)

        drain(1, sem1)
        accumulate(b1, 1)
        return 0

    lax.fori_loop(0, BPW // 2, body, 0)
    pltpu.sync_copy(out_v, out_hbm.at[pl.ds(base * D, BPW * D)])


def _bag_means(idxp, parp, table2):
    mesh = plsc.VectorSubcoreMesh(core_axis_name="c", subcore_axis_name="s")
    f = functools.partial(
        pl.kernel,
        mesh=mesh,
        out_type=jax.ShapeDtypeStruct((B * D,), jnp.float32),
        scratch_types=[
            pltpu.VMEM((BPW * PSEQ,), jnp.int32),
            pltpu.VMEM((BPW * PSEQ,), jnp.int32),
            pltpu.VMEM((2, SEQ, W), jnp.float32),
            pltpu.VMEM((BPW * D,), jnp.float32),
            pltpu.SemaphoreType.DMA,
            pltpu.SemaphoreType.DMA,
        ],
    )(_bag_body)
    return f(idxp, parp, table2)


def _mlp_body(bag_ref, w1_ref, b1_ref, w2_ref, b2_ref, out_ref):
    h = jnp.dot(bag_ref[...], w1_ref[...], preferred_element_type=jnp.float32)
    h = jnp.maximum(h + b1_ref[...], 0.0)
    out_ref[...] = (
        jnp.dot(h, w2_ref[...], preferred_element_type=jnp.float32)
        + b2_ref[...]
    )


def _mlp(bag, W1, b1, W2, b2):
    return pl.pallas_call(
        _mlp_body,
        out_shape=jax.ShapeDtypeStruct((B, C), jnp.float32),
    )(bag, W1, b1.reshape(1, H), W2, b2.reshape(1, C))


def kernel(inputX, emb, W1, b1, W2, b2):
    idx = inputX.astype(jnp.int32)
    idxp = jnp.pad(idx % VROWS, ((0, 0), (0, PSEQ - SEQ))).reshape(-1)
    parp = jnp.pad(
        (idx >= VROWS).astype(jnp.int32), ((0, 0), (0, PSEQ - SEQ))
    ).reshape(-1)
    table2 = _pack_table(emb.T)
    bag = _bag_means(idxp, parp, table2).reshape(B, D)
    return _mlp(bag, W1, b1, W2, b2)


# speedup vs baseline: 2.0935x; 1.0351x over previous
import functools

import jax
import jax.numpy as jnp
from jax import lax
from jax.experimental import pallas as pl
from jax.experimental.pallas import tpu as pltpu
from jax.experimental.pallas import tpu_sc as plsc

VOCAB = 1000000
D = 64
W = 128
TBLK = 16384
TGRID = 31
VROWS = TBLK * TGRID
SEQ = 200
PSEQ = 256
NFULL = SEQ // 16
TAIL = SEQ - 16 * NFULL
B = 4096
H = 100
C = 10

_info = plsc.get_sparse_core_info()
NC = _info.num_cores
NS = _info.num_subcores
NW = NC * NS
BPW = B // NW
NV = D // 16


def _pack_body(lo_ref, hi_ref, out_ref):
    out_ref[:, 0:D] = jnp.transpose(lo_ref[...])
    out_ref[:, D:W] = jnp.transpose(hi_ref[...])


def _pack_table(embT):
    return pl.pallas_call(
        _pack_body,
        grid=(TGRID,),
        in_specs=[
            pl.BlockSpec((D, TBLK), lambda j: (0, j)),
            pl.BlockSpec(
                (D, TBLK),
                lambda j: (0, jnp.minimum(j + TGRID, VOCAB // TBLK)),
            ),
        ],
        out_specs=pl.BlockSpec((TBLK, W), lambda j: (j, 0)),
        out_shape=jax.ShapeDtypeStruct((VROWS, W), jnp.float32),
    )(embT, embT)


def _bag_body(idx_hbm, par_hbm, table_hbm, out_hbm,
              idx_v, par_v, rows_v, out_v, sem0, sem1):
    wid = lax.axis_index("s") * NC + lax.axis_index("c")
    base = wid * BPW

    pltpu.sync_copy(idx_hbm.at[pl.ds(base * PSEQ, BPW * PSEQ)], idx_v)
    pltpu.sync_copy(par_hbm.at[pl.ds(base * PSEQ, BPW * PSEQ)], par_v)

    def issue(b, buf, sem):
        pltpu.async_copy(
            table_hbm.at[idx_v.at[pl.ds(b * PSEQ, 128)]],
            rows_v.at[buf, pl.ds(0, 128)], sem)
        pltpu.async_copy(
            table_hbm.at[idx_v.at[pl.ds(b * PSEQ + 128, 72)]],
            rows_v.at[buf, pl.ds(128, 72)], sem)

    def drain(buf, sem):
        pltpu.make_async_copy(
            table_hbm.at[pl.ds(0, 128)], rows_v.at[buf, pl.ds(0, 128)],
            sem).wait()
        pltpu.make_async_copy(
            table_hbm.at[pl.ds(0, 72)], rows_v.at[buf, pl.ds(128, 72)],
            sem).wait()

    def accumulate(b, buf):
        def rows16(g, accs, n=16):
            pv = par_v[pl.ds(b * PSEQ + g * 16, 16)]
            for j in range(n):
                r = g * 16 + j
                hi_half = pv[j] != 0
                new = []
                for c in range(NV):
                    lo = rows_v[buf, r, pl.ds(c * 16, 16)]
                    hi = rows_v[buf, r, pl.ds(D + c * 16, 16)]
                    new.append(accs[c] + jnp.where(hi_half, hi, lo))
                accs = tuple(new)
            return accs

        zero = jnp.zeros((16,), jnp.float32)
        accs = lax.fori_loop(0, NFULL, rows16, (zero,) * NV)
        accs = rows16(NFULL, accs, n=TAIL)
        inv = jnp.float32(1.0 / SEQ)
        for c in range(NV):
            out_v[pl.ds(b * D + c * 16, 16)] = accs[c] * inv

    issue(0, 0, sem0)

    def body(i, _):
        b0 = 2 * i
        b1 = 2 * i + 1
        issue(b1, 1, sem1)
        drain(0, sem0)
        accumulate(b0, 0)

        @pl.when(b1 + 1 < BPW)
        def _():
            issue(b1 + 1, 0, sem0)

        drain(1, sem1)
        accumulate(b1, 1)
        return 0

    lax.fori_loop(0, BPW // 2, body, 0)
    pltpu.sync_copy(out_v, out_hbm.at[pl.ds(base * D, BPW * D)])


def _bag_means(idxp, parp, table2):
    mesh = plsc.VectorSubcoreMesh(core_axis_name="c", subcore_axis_name="s")
    f = functools.partial(
        pl.kernel,
        mesh=mesh,
        out_type=jax.ShapeDtypeStruct((B * D,), jnp.float32),
        scratch_types=[
            pltpu.VMEM((BPW * PSEQ,), jnp.int32),
            pltpu.VMEM((BPW * PSEQ,), jnp.int32),
            pltpu.VMEM((2, SEQ, W), jnp.float32),
            pltpu.VMEM((BPW * D,), jnp.float32),
            pltpu.SemaphoreType.DMA,
            pltpu.SemaphoreType.DMA,
        ],
    )(_bag_body)
    return f(idxp, parp, table2)


def _mlp_body(bag_ref, w1_ref, b1_ref, w2_ref, b2_ref, out_ref):
    h = jnp.dot(bag_ref[...], w1_ref[...], preferred_element_type=jnp.float32)
    h = jnp.maximum(h + b1_ref[...], 0.0)
    out_ref[...] = (
        jnp.dot(h, w2_ref[...], preferred_element_type=jnp.float32)
        + b2_ref[...]
    )


def _mlp(bag, W1, b1, W2, b2):
    return pl.pallas_call(
        _mlp_body,
        out_shape=jax.ShapeDtypeStruct((B, C), jnp.float32),
    )(bag, W1, b1.reshape(1, H), W2, b2.reshape(1, C))


def kernel(inputX, emb, W1, b1, W2, b2):
    idx = inputX.astype(jnp.int32)
    idxp = jnp.pad(idx % VROWS, ((0, 0), (0, PSEQ - SEQ))).reshape(-1)
    parp = jnp.pad(
        (idx >= VROWS).astype(jnp.int32), ((0, 0), (0, PSEQ - SEQ))
    ).reshape(-1)
    table2 = _pack_table(emb.T)
    bag = _bag_means(idxp, parp, table2).reshape(B, D)
    return _mlp(bag, W1, b1, W2, b2)
